# baseline (speedup 1.0000x reference)
import functools

import numpy as np
import jax
import jax.numpy as jnp
from jax import lax
from jax.experimental import pallas as pl
from jax.experimental.pallas import tpu as pltpu
from jax.experimental.pallas import tpu_sc as plsc

_N, _C, _H, _W = 32, 384, 56, 56
_D = _H * _W
_B = _N * _C

_PERM = np.asarray(jax.random.permutation(jax.random.key(42), _C))
_ROW_IDX = (
    np.arange(_N, dtype=np.int64)[:, None] * _C + _PERM[None, :]
).reshape(-1).astype(np.int32)

_NC, _NS = 2, 16
_NW = _NC * _NS
_RPW = _B // _NW
_CH = 16
_NCHUNK = _RPW // _CH

_mesh = plsc.VectorSubcoreMesh(core_axis_name="c", subcore_axis_name="s")


@functools.partial(
    pl.kernel,
    mesh=_mesh,
    out_type=jax.ShapeDtypeStruct((_B, _D), jnp.float32),
    compiler_params=pltpu.CompilerParams(use_tc_tiling_on_sc=False),
    scratch_types=[
        pltpu.VMEM((_RPW,), jnp.int32),
        pltpu.VMEM((_CH, _D), jnp.float32),
        pltpu.SemaphoreType.DMA,
    ],
)
def _shuffle_rows(x_hbm, idx_hbm, out_hbm, idx_v, buf, sem):
    wid = lax.axis_index("s") * _NC + lax.axis_index("c")
    base = wid * _RPW
    pltpu.sync_copy(idx_hbm.at[pl.ds(base, _RPW)], idx_v)

    def body(i, carry):
        off = i * _CH
        pltpu.async_copy(x_hbm.at[idx_v.at[pl.ds(off, _CH)]], buf, sem).wait()
        pltpu.sync_copy(buf, out_hbm.at[pl.ds(base + off, _CH)])
        return carry

    lax.fori_loop(0, _NCHUNK, body, 0)


def kernel(input):
    x2 = input.reshape(_B, _D)
    out2 = _shuffle_rows(x2, jnp.asarray(_ROW_IDX))
    return out2.reshape(_N, _C, _H, _W)


# speedup vs baseline: 1.0122x; 1.0122x over previous
import functools

import numpy as np
import jax
import jax.numpy as jnp
from jax import lax
from jax.experimental import pallas as pl
from jax.experimental.pallas import tpu as pltpu
from jax.experimental.pallas import tpu_sc as plsc

_N, _C, _H, _W = 32, 384, 56, 56
_D = _H * _W
_B = _N * _C

_PERM = np.asarray(jax.random.permutation(jax.random.key(42), _C))
_ROW_IDX = (
    np.arange(_N, dtype=np.int64)[:, None] * _C + _PERM[None, :]
).reshape(-1).astype(np.int32)

_NC, _NS = 2, 16
_NW = _NC * _NS
_RPW = _B // _NW
_CH = 16
_NCHUNK = _RPW // _CH

_mesh = plsc.VectorSubcoreMesh(core_axis_name="c", subcore_axis_name="s")


@functools.partial(
    pl.kernel,
    mesh=_mesh,
    out_type=jax.ShapeDtypeStruct((_B, _D), jnp.float32),
    compiler_params=pltpu.CompilerParams(use_tc_tiling_on_sc=False),
    scratch_types=[
        pltpu.VMEM((_RPW,), jnp.int32),
        pltpu.VMEM((_CH, _D), jnp.float32),
        pltpu.VMEM((_CH, _D), jnp.float32),
        pltpu.SemaphoreType.DMA,
        pltpu.SemaphoreType.DMA,
        pltpu.SemaphoreType.DMA,
        pltpu.SemaphoreType.DMA,
    ],
)
def _shuffle_rows(x_hbm, idx_hbm, out_hbm, idx_v, buf0, buf1, g0, g1, s0, s1):
    bufs, gsems, ssems = (buf0, buf1), (g0, g1), (s0, s1)
    wid = lax.axis_index("s") * _NC + lax.axis_index("c")
    base = wid * _RPW
    pltpu.sync_copy(idx_hbm.at[pl.ds(base, _RPW)], idx_v)

    def g_copy(v, b):
        return pltpu.make_async_copy(
            x_hbm.at[idx_v.at[pl.ds(v * _CH, _CH)]], bufs[b], gsems[b])

    def s_copy(v, b):
        return pltpu.make_async_copy(
            bufs[b], out_hbm.at[pl.ds(base + v * _CH, _CH)], ssems[b])

    g_copy(0, 0).start()

    @pl.loop(0, _NCHUNK, step=2)
    def _(i):
        for b in range(2):
            v = i + b
            g_copy(v, b).wait()
            s_copy(v, b).start()

            @pl.when(v + 1 < _NCHUNK)
            def _start_next():
                @pl.when(v >= 1)
                def _drain_prev():
                    s_copy(v - 1, 1 - b).wait()
                g_copy(v + 1, 1 - b).start()

    s_copy(_NCHUNK - 2, 0).wait()
    s_copy(_NCHUNK - 1, 1).wait()


def kernel(input):
    x2 = input.reshape(_B, _D)
    out2 = _shuffle_rows(x2, jnp.asarray(_ROW_IDX))
    return out2.reshape(_N, _C, _H, _W)
